# baseline (speedup 1.0000x reference)
import functools

import jax
import jax.numpy as jnp
from jax import lax
from jax.experimental import pallas as pl
from jax.experimental.pallas import tpu as pltpu
from jax.experimental.pallas import tpu_sc as plsc

N = 10000
E = 320000
D = 128
G = 64

NC = 2
NS = 16
NW = NC * NS
CH = 125
NCHUNK = 80
BLKC = 16
N_PAD = 10240
RPT = N_PAD // NS

_mesh = functools.partial(
    plsc.VectorSubcoreMesh,
    core_axis_name="c", subcore_axis_name="s", num_cores=NC, num_subcores=NS,
)


def _sc_body(src_hbm, dst_hbm, x_hbm, u_hbm, ys_hbm, dinv_hbm,
             src_slab, dst_blk, rows, dvec, ones, shu, shdeg, sem_g, sem_d):
    c = lax.axis_index("c")
    s = lax.axis_index("s")
    w = c * NS + s
    r0 = s * RPT

    def _z(i, _):
        rows[i // 8, pl.ds((i % 8) * 16, 16)] = jnp.zeros((16,), jnp.float32)
        return 0
    lax.fori_loop(0, 128 * 8, _z, 0)

    def _zd(i, _):
        dvec[pl.ds(i * 16, 16)] = jnp.zeros((16,), jnp.float32)
        return 0
    lax.fori_loop(0, RPT // 16, _zd, 0)

    def _o(i, _):
        ones[pl.ds(i * 16, 16)] = jnp.ones((16,), jnp.float32)
        return 0
    lax.fori_loop(0, 128 // 16, _o, 0)

    def _zs(bi, _):
        pltpu.sync_copy(rows.at[pl.ds(0, 128)],
                        shu.at[pl.ds(r0 + bi * 128, 128)])
        return 0
    lax.fori_loop(0, RPT // 128, _zs, 0)
    pltpu.sync_copy(dvec, shdeg.at[pl.ds(r0, RPT)])
    plsc.subcore_barrier()

    for half in range(NC):
        w2 = half * NS + s
        for bb in range(NCHUNK // BLKC):
            pltpu.sync_copy(dst_hbm.at[w2, pl.ds(bb * BLKC, BLKC)],
                            dst_blk.at[0])

            def _acc(j, _):
                pltpu.sync_copy(ones.at[pl.ds(0, CH)],
                                shdeg.at[dst_blk.at[0, j]], add=True)
                return 0
            lax.fori_loop(0, BLKC, _acc, 0)
    plsc.subcore_barrier()

    pltpu.sync_copy(shdeg.at[pl.ds(r0, RPT)], dvec)

    def _rsq(g, _):
        d = dvec[pl.ds(g * 16, 16)] + 1.0
        i0 = lax.bitcast_convert_type(d, jnp.int32)
        r = lax.bitcast_convert_type(jnp.int32(0x5F3759DF) - (i0 >> 1),
                                     jnp.float32)
        r = r * (1.5 - 0.5 * d * r * r)
        r = r * (1.5 - 0.5 * d * r * r)
        r = r * (1.5 - 0.5 * d * r * r)
        dvec[pl.ds(g * 16, 16)] = r
        return 0
    lax.fori_loop(0, RPT // 16, _rsq, 0)

    @pl.when(c == 0)
    def _():
        pltpu.sync_copy(dvec, dinv_hbm.at[pl.ds(r0, RPT)])

    for ch_ in range(RPT // 128):
        base = r0 + ch_ * 128
        pltpu.sync_copy(x_hbm.at[pl.ds(base, 128)], rows.at[pl.ds(0, 128)])

        def _row(row, _):
            dv16 = dvec[pl.ds(ch_ * 128 + (row // 16) * 16, 16)]
            bc = dv16[jnp.broadcast_to(row % 16, (16,)).astype(jnp.int32)]
            for k in range(8):
                rows[128 + row, pl.ds(k * 16, 16)] = (
                    rows[row, pl.ds(k * 16, 16)] * bc)
            return 0
        lax.fori_loop(0, 128, _row, 0)
        pltpu.sync_copy(rows.at[pl.ds(128, 128)],
                        ys_hbm.at[pl.ds(c * N_PAD + base, 128)])
    plsc.subcore_barrier()

    def _gather(j, b):
        return pltpu.make_async_copy(
            ys_hbm.at[src_slab.at[j]], rows.at[pl.ds(b * 128, CH)],
            sem_g.at[b])

    pltpu.sync_copy(src_hbm.at[w], src_slab)

    pltpu.make_async_copy(dst_hbm.at[w, pl.ds(0, BLKC)], dst_blk.at[0],
                          sem_d.at[0]).start()
    _gather(0, 0).start()
    _gather(1, 1).start()

    for bb in range(NCHUNK // BLKC):
        p = bb % 2
        pltpu.make_async_copy(dst_hbm.at[w, pl.ds(bb * BLKC, BLKC)],
                              dst_blk.at[p], sem_d.at[p]).wait()
        if bb + 1 < NCHUNK // BLKC:
            pltpu.make_async_copy(
                dst_hbm.at[w, pl.ds((bb + 1) * BLKC, BLKC)],
                dst_blk.at[1 - p], sem_d.at[1 - p]).start()

        def _pair(k, _2):
            for b in range(2):
                j = bb * BLKC + k * 2 + b
                _gather(j, b).wait()
                pltpu.sync_copy(rows.at[pl.ds(b * 128, CH)],
                                shu.at[dst_blk.at[p, k * 2 + b]], add=True)

                @pl.when(j + 2 < NCHUNK)
                def _():
                    _gather(j + 2, b).start()
            return 0
        lax.fori_loop(0, BLKC // 2, _pair, 0)

    plsc.subcore_barrier()

    def _out(bi, _):
        rr = r0 + bi * 128
        pltpu.sync_copy(shu.at[pl.ds(rr, 128)], u_hbm.at[c, pl.ds(rr, 128)])
        return 0
    lax.fori_loop(0, RPT // 128, _out, 0)


def _sc_call(src3, dst3, x_pad):
    k = pl.kernel(
        _sc_body,
        out_type=[
            jax.ShapeDtypeStruct((NC, N_PAD, D), jnp.float32),
            jax.ShapeDtypeStruct((NC * N_PAD, D), jnp.float32),
            jax.ShapeDtypeStruct((N_PAD,), jnp.float32),
        ],
        mesh=_mesh(),
        scratch_types=[
            pltpu.VMEM((NCHUNK, CH), jnp.int32),
            pltpu.VMEM((2, BLKC, CH), jnp.int32),
            pltpu.VMEM((2 * 128, D), jnp.float32),
            pltpu.VMEM((RPT,), jnp.float32),
            pltpu.VMEM((128,), jnp.float32),
            pltpu.VMEM_SHARED((N_PAD, D), jnp.float32),
            pltpu.VMEM_SHARED((N_PAD,), jnp.float32),
            pltpu.SemaphoreType.DMA((2,)),
            pltpu.SemaphoreType.DMA((2,)),
        ],
    )
    return k(src3, dst3, x_pad)


_BN = 2000
_NBLK = N // _BN


def _final_body(u0_ref, u1_ref, y_ref, dinv_ref, batch_ref,
                w_ref, b_ref, fcw_ref, fcb_ref, out_ref, num_acc, cnt_acc):
    i = pl.program_id(0)

    @pl.when(i == 0)
    def _():
        num_acc[...] = jnp.zeros((1, G), jnp.float32)
        cnt_acc[...] = jnp.zeros((1, G), jnp.float32)

    z = (u0_ref[0] + u1_ref[0] + y_ref[...]) * dinv_ref[...]
    h = jnp.dot(z, w_ref[...]) + b_ref[...]
    h = jnp.maximum(h, 0.0)
    t = jnp.dot(h, fcw_ref[...])

    gids = lax.broadcasted_iota(jnp.int32, (1, G), 1)
    m = (batch_ref[...] == gids).astype(jnp.float32)
    num_acc[...] += jnp.sum(m * t, axis=0, keepdims=True)
    cnt_acc[...] += jnp.sum(m, axis=0, keepdims=True)

    @pl.when(i == _NBLK - 1)
    def _():
        pooled = num_acc[...] / jnp.maximum(cnt_acc[...], 1.0)
        logits = pooled + fcb_ref[...]
        out_ref[...] = jax.nn.sigmoid(logits)


def _final_call(u2, y, dinv, batch2d, W, b2d, fc_W, fc_b2d):
    row = lambda i: (i, 0)
    fixed = lambda i: (0, 0)
    out = pl.pallas_call(
        _final_body,
        grid=(_NBLK,),
        in_specs=[
            pl.BlockSpec((1, _BN, D), lambda i: (0, i, 0)),
            pl.BlockSpec((1, _BN, D), lambda i: (1, i, 0)),
            pl.BlockSpec((_BN, D), row),
            pl.BlockSpec((_BN, 1), row),
            pl.BlockSpec((_BN, 1), row),
            pl.BlockSpec((D, D), fixed),
            pl.BlockSpec((1, D), fixed),
            pl.BlockSpec((D, 1), fixed),
            pl.BlockSpec((1, 1), fixed),
        ],
        out_specs=pl.BlockSpec((1, G), fixed),
        out_shape=jax.ShapeDtypeStruct((1, G), jnp.float32),
        scratch_shapes=[
            pltpu.VMEM((1, G), jnp.float32),
            pltpu.VMEM((1, G), jnp.float32),
        ],
    )(u2, u2, y, dinv, batch2d, W, b2d, fc_W, fc_b2d)
    return out


def kernel(x, W, b, fc_W, fc_b, edge_index, batch):
    off = (jnp.arange(NW, dtype=jnp.int32)[:, None, None] // NS) * N_PAD
    src3 = edge_index[0].reshape(NW, NCHUNK, CH) + off
    dst3 = edge_index[1].reshape(NW, NCHUNK, CH)
    x_pad = jnp.pad(x, ((0, N_PAD - N), (0, 0)))

    u2, ys, dinv = _sc_call(src3, dst3, x_pad)

    out = _final_call(u2, ys[:N], dinv[:N, None], batch.reshape(N, 1),
                      W, b.reshape(1, D), fc_W, fc_b.reshape(1, 1))
    return out.reshape(G, 1)


# speedup vs baseline: 1.2191x; 1.2191x over previous
import functools

import jax
import jax.numpy as jnp
from jax import lax
from jax.experimental import pallas as pl
from jax.experimental.pallas import tpu as pltpu
from jax.experimental.pallas import tpu_sc as plsc

N = 10000
E = 320000
D = 128
G = 64

NC = 2
NS = 16
NW = NC * NS
CH = 125
NCHUNK = 80
BLKC = 16
N_PAD = 10240
RPT = N_PAD // NS

_mesh = functools.partial(
    plsc.VectorSubcoreMesh,
    core_axis_name="c", subcore_axis_name="s", num_cores=NC, num_subcores=NS,
)


def _deg_body(dst_hbm, out_hbm, slab, ones, zbuf, shdeg, sem):
    c = lax.axis_index("c")
    s = lax.axis_index("s")

    def _z(i, _):
        zbuf[pl.ds(i * 16, 16)] = jnp.zeros((16,), jnp.float32)
        return 0
    lax.fori_loop(0, RPT // 16, _z, 0)

    def _o(i, _):
        ones[pl.ds(i * 16, 16)] = jnp.ones((16,), jnp.float32)
        return 0
    lax.fori_loop(0, 128 // 16, _o, 0)

    pltpu.sync_copy(zbuf, shdeg.at[pl.ds(s * RPT, RPT)])
    plsc.subcore_barrier()

    w = c * NS + s
    pltpu.sync_copy(dst_hbm.at[w], slab)

    def _acc(j, _):
        pltpu.async_copy(ones.at[pl.ds(0, CH)], shdeg.at[slab.at[j]],
                         sem, add=True)
        return 0
    lax.fori_loop(0, NCHUNK, _acc, 0)

    def _dr(j, _):
        pltpu.make_async_copy(ones.at[pl.ds(0, CH)], shdeg.at[slab.at[0]],
                              sem).wait()
        return 0
    lax.fori_loop(0, NCHUNK, _dr, 0)

    plsc.subcore_barrier()
    pltpu.sync_copy(shdeg.at[pl.ds(s * RPT, RPT)],
                    out_hbm.at[c, pl.ds(s * RPT, RPT)])


def _deg_call(dst3):
    k = pl.kernel(
        _deg_body,
        out_type=jax.ShapeDtypeStruct((NC, N_PAD), jnp.float32),
        mesh=_mesh(),
        scratch_types=[
            pltpu.VMEM((NCHUNK, CH), jnp.int32),
            pltpu.VMEM((128,), jnp.float32),
            pltpu.VMEM((RPT,), jnp.float32),
            pltpu.VMEM_SHARED((N_PAD,), jnp.float32),
            pltpu.SemaphoreType.DMA,
        ],
    )
    return k(dst3)


def _prescale_body(x_ref, d_ref, y_ref, dinv_ref):
    deg = d_ref[0] + d_ref[1] + 1.0
    dinv = lax.rsqrt(deg)
    dinv_ref[...] = dinv
    y_ref[...] = x_ref[...] * dinv


def _prescale_call(x, deg_parts):
    return pl.pallas_call(
        _prescale_body,
        grid=(1,),
        in_specs=[
            pl.BlockSpec((N, D), lambda i: (0, 0)),
            pl.BlockSpec((NC, N, 1), lambda i: (0, 0, 0)),
        ],
        out_specs=[
            pl.BlockSpec((N, D), lambda i: (0, 0)),
            pl.BlockSpec((N, 1), lambda i: (0, 0)),
        ],
        out_shape=[
            jax.ShapeDtypeStruct((N, D), jnp.float32),
            jax.ShapeDtypeStruct((N, 1), jnp.float32),
        ],
    )(x, deg_parts)


def _scatter_body(src_hbm, dst_hbm, y_hbm, out_hbm,
                  src_slab, dst_blk, rows, shu, sem_g, sem_d):
    c = lax.axis_index("c")
    s = lax.axis_index("s")
    w = c * NS + s

    def _gather(j, b):
        return pltpu.make_async_copy(
            y_hbm.at[src_slab.at[j]], rows.at[pl.ds(b * 128, CH)],
            sem_g.at[b])

    def _z(i, _):
        rows[i // 8, pl.ds((i % 8) * 16, 16)] = jnp.zeros((16,), jnp.float32)
        return 0
    lax.fori_loop(0, 128 * 8, _z, 0)

    def _zs(bi, _):
        pltpu.sync_copy(rows.at[pl.ds(0, 128)],
                        shu.at[pl.ds(s * RPT + bi * 128, 128)])
        return 0
    lax.fori_loop(0, RPT // 128, _zs, 0)
    plsc.subcore_barrier()

    pltpu.sync_copy(src_hbm.at[w], src_slab)
    pltpu.make_async_copy(dst_hbm.at[w, pl.ds(0, BLKC)], dst_blk.at[0],
                          sem_d.at[0]).start()
    _gather(0, 0).start()
    _gather(1, 1).start()

    for bb in range(NCHUNK // BLKC):
        p = bb % 2
        pltpu.make_async_copy(dst_hbm.at[w, pl.ds(bb * BLKC, BLKC)],
                              dst_blk.at[p], sem_d.at[p]).wait()
        if bb + 1 < NCHUNK // BLKC:
            pltpu.make_async_copy(
                dst_hbm.at[w, pl.ds((bb + 1) * BLKC, BLKC)],
                dst_blk.at[1 - p], sem_d.at[1 - p]).start()

        def _pair(k, _2):
            for b in range(2):
                j = bb * BLKC + k * 2 + b
                _gather(j, b).wait()
                pltpu.sync_copy(rows.at[pl.ds(b * 128, CH)],
                                shu.at[dst_blk.at[p, k * 2 + b]], add=True)

                @pl.when(j + 2 < NCHUNK)
                def _():
                    _gather(j + 2, b).start()
            return 0
        lax.fori_loop(0, BLKC // 2, _pair, 0)

    plsc.subcore_barrier()

    def _out(bi, _):
        r0 = s * RPT + bi * 128
        pltpu.sync_copy(shu.at[pl.ds(r0, 128)], out_hbm.at[c, pl.ds(r0, 128)])
        return 0
    lax.fori_loop(0, RPT // 128, _out, 0)


def _scatter_call(src3, dst3, y):
    k = pl.kernel(
        _scatter_body,
        out_type=jax.ShapeDtypeStruct((NC, N_PAD, D), jnp.float32),
        mesh=_mesh(),
        scratch_types=[
            pltpu.VMEM((NCHUNK, CH), jnp.int32),
            pltpu.VMEM((2, BLKC, CH), jnp.int32),
            pltpu.VMEM((2 * 128, D), jnp.float32),
            pltpu.VMEM_SHARED((N_PAD, D), jnp.float32),
            pltpu.SemaphoreType.DMA((2,)),
            pltpu.SemaphoreType.DMA((2,)),
        ],
    )
    return k(src3, dst3, y)


_BN = 2000
_NBLK = N // _BN


def _final_body(u0_ref, u1_ref, y_ref, dinv_ref, batch_ref,
                w_ref, b_ref, fcw_ref, fcb_ref, out_ref, num_acc, cnt_acc):
    i = pl.program_id(0)

    @pl.when(i == 0)
    def _():
        num_acc[...] = jnp.zeros((1, G), jnp.float32)
        cnt_acc[...] = jnp.zeros((1, G), jnp.float32)

    z = (u0_ref[0] + u1_ref[0] + y_ref[...]) * dinv_ref[...]
    h = jnp.dot(z, w_ref[...]) + b_ref[...]
    h = jnp.maximum(h, 0.0)
    t = jnp.dot(h, fcw_ref[...])

    gids = lax.broadcasted_iota(jnp.int32, (1, G), 1)
    m = (batch_ref[...] == gids).astype(jnp.float32)
    num_acc[...] += jnp.sum(m * t, axis=0, keepdims=True)
    cnt_acc[...] += jnp.sum(m, axis=0, keepdims=True)

    @pl.when(i == _NBLK - 1)
    def _():
        pooled = num_acc[...] / jnp.maximum(cnt_acc[...], 1.0)
        logits = pooled + fcb_ref[...]
        out_ref[...] = jax.nn.sigmoid(logits)


def _final_call(u2, y, dinv, batch2d, W, b2d, fc_W, fc_b2d):
    row = lambda i: (i, 0)
    fixed = lambda i: (0, 0)
    out = pl.pallas_call(
        _final_body,
        grid=(_NBLK,),
        in_specs=[
            pl.BlockSpec((1, _BN, D), lambda i: (0, i, 0)),
            pl.BlockSpec((1, _BN, D), lambda i: (1, i, 0)),
            pl.BlockSpec((_BN, D), row),
            pl.BlockSpec((_BN, 1), row),
            pl.BlockSpec((_BN, 1), row),
            pl.BlockSpec((D, D), fixed),
            pl.BlockSpec((1, D), fixed),
            pl.BlockSpec((D, 1), fixed),
            pl.BlockSpec((1, 1), fixed),
        ],
        out_specs=pl.BlockSpec((1, G), fixed),
        out_shape=jax.ShapeDtypeStruct((1, G), jnp.float32),
        scratch_shapes=[
            pltpu.VMEM((1, G), jnp.float32),
            pltpu.VMEM((1, G), jnp.float32),
        ],
    )(u2, u2, y, dinv, batch2d, W, b2d, fc_W, fc_b2d)
    return out


def kernel(x, W, b, fc_W, fc_b, edge_index, batch):
    src3 = edge_index[0].reshape(NW, NCHUNK, CH)
    dst3 = edge_index[1].reshape(NW, NCHUNK, CH)

    deg_parts = _deg_call(dst3)
    y, dinv = _prescale_call(x, deg_parts.reshape(NC, N_PAD, 1))

    u2 = _scatter_call(src3, dst3, y)

    out = _final_call(u2, y, dinv, batch.reshape(N, 1),
                      W, b.reshape(1, D), fc_W, fc_b.reshape(1, 1))
    return out.reshape(G, 1)
